# baseline (speedup 1.0000x reference)
import jax
import jax.numpy as jnp
from jax.experimental import pallas as pl
from jax.experimental.pallas import tpu as pltpu

_CELL = 3.0
_BB = 64


def _relu(x):
    return jnp.maximum(x, 0.0)


def _body_a(st_ref, xl_ref, yl_ref, xs_ref, ys_ref, self6_ref,
            w1a, b1a, w1b, b1b, w2a, b2a, w2b, b2b,
            wa0t, wa0g, ba0, wa1, ba1, wa2v, ba2, joint_ref):
    st = st_ref[...]
    BB, N, _ = st.shape

    xk = xl_ref[...][:, None, :]
    yk = yl_ref[...][:, None, :]
    xj = xs_ref[...]
    yj = ys_ref[...]
    dx = xk - xj
    dy = yk - yj
    dist = jnp.sqrt(dx * dx + dy * dy)
    w = jnp.where(dist < _CELL, 1.0, 0.0)
    wx = jnp.where(dx > 0.0, w, 0.0)
    wy = jnp.where(dy > 0.0, w, 0.0)
    wxy = jnp.where(dy > 0.0, wx, 0.0)
    c0 = jnp.sum(wxy, axis=2, keepdims=True)
    sx = jnp.sum(wx, axis=2, keepdims=True)
    sy = jnp.sum(wy, axis=2, keepdims=True)
    sw = jnp.sum(w, axis=2, keepdims=True)
    c1 = sx - c0
    c2 = sy - c0
    c3 = sw - sx - sy + c0 - 1.0
    counts = jnp.concatenate([c0, c1, c2, c3], axis=-1)

    s3 = jnp.concatenate([st, counts], axis=-1)
    x2 = s3.reshape(BB * N, 17)

    h = _relu(jnp.dot(x2, w1a[...]) + b1a[...])
    m1 = _relu(jnp.dot(h, w1b[...]) + b1b[...])
    h = _relu(jnp.dot(m1, w2a[...]) + b2a[...])
    m2 = jnp.dot(h, w2b[...]) + b2b[...]

    g = jnp.mean(m1.reshape(BB, N, 100), axis=1)
    gb = jnp.dot(g, wa0g[...])
    gb = jnp.broadcast_to(gb[:, None, :], (BB, N, 100)).reshape(BB * N, 100)
    ha = _relu(jnp.dot(m1, wa0t[...]) + gb + ba0[...])
    hb = _relu(jnp.dot(ha, wa1[...]) + ba1[...])
    scores = jnp.sum(hb.reshape(BB, N, 100) * wa2v[...][None, :, :], axis=-1)
    scores = scores + ba2[...]

    e = jnp.where(scores != 0.0, jnp.exp(scores), 0.0)
    wts = e / jnp.sum(e, axis=-1, keepdims=True)
    weighted = jnp.sum(wts[:, :, None] * m2.reshape(BB, N, 50), axis=1)

    joint_ref[...] = jnp.concatenate([self6_ref[...], weighted], axis=-1)


def _body_b(joint_ref, w30, b30, w31, b31, w32, b32, w33, b33,
            wA0, bA0, wA1, bA1, wA2, bA2, wS0, bS0, wS1, bS1, out_ref):
    j = joint_ref[...]
    v = _relu(jnp.dot(j, w30[...]) + b30[...])
    v = _relu(jnp.dot(v, w31[...]) + b31[...])
    v = _relu(jnp.dot(v, w32[...]) + b32[...])
    value = jnp.dot(v, w33[...]) + b33[...]
    a = _relu(jnp.dot(j, wA0[...]) + bA0[...])
    a = _relu(jnp.dot(a, wA1[...]) + bA1[...])
    a = jnp.dot(a, wA2[...]) + bA2[...]
    s = _relu(jnp.dot(a, wS0[...]) + bS0[...])
    adv = jnp.dot(s, wS1[...]) + bS1[...]
    out_ref[...] = value + adv - jnp.mean(adv)


def _full(w_and_b):
    W, b = w_and_b
    return [W, b.reshape(1, -1)]


def kernel(state, params):
    B, N, D = state.shape
    p1, p2, pa = params['mlp1'], params['mlp2'], params['attention']
    p3, p3a, ps = params['mlp3'], params['mlp3a'], params['adv_stream']

    xl = state[:, :, 6]
    yl = state[:, :, 7]
    xs = xl[:, :, None]
    ys = yl[:, :, None]
    self6 = state[:, 0, :6]

    wa0, ba0 = pa[0]
    flat_a = (_full(p1[0]) + _full(p1[1]) + _full(p2[0]) + _full(p2[1])
              + [wa0[:100, :], wa0[100:, :], ba0.reshape(1, -1)]
              + _full(pa[1])
              + [pa[2][0].reshape(1, -1), pa[2][1].reshape(1, 1)])

    grid = (B // _BB,)
    full = lambda a: pl.BlockSpec(a.shape, lambda i: (0,) * a.ndim)
    joint = pl.pallas_call(
        _body_a,
        grid=grid,
        in_specs=[pl.BlockSpec((_BB, N, D), lambda i: (i, 0, 0)),
                  pl.BlockSpec((_BB, N), lambda i: (i, 0)),
                  pl.BlockSpec((_BB, N), lambda i: (i, 0)),
                  pl.BlockSpec((_BB, N, 1), lambda i: (i, 0, 0)),
                  pl.BlockSpec((_BB, N, 1), lambda i: (i, 0, 0)),
                  pl.BlockSpec((_BB, 6), lambda i: (i, 0))]
        + [full(a) for a in flat_a],
        out_specs=pl.BlockSpec((_BB, 56), lambda i: (i, 0)),
        out_shape=jax.ShapeDtypeStruct((B, 56), jnp.float32),
        compiler_params=pltpu.CompilerParams(
            dimension_semantics=("parallel",),
            vmem_limit_bytes=100 * 1024 * 1024,
        ),
        name="value_net_main",
    )(state, xl, yl, xs, ys, self6, *flat_a)

    flat_b = (_full(p3[0]) + _full(p3[1]) + _full(p3[2]) + _full(p3[3])
              + _full(p3a[0]) + _full(p3a[1]) + _full(p3a[2])
              + _full(ps[0]) + _full(ps[1]))
    out = pl.pallas_call(
        _body_b,
        out_shape=jax.ShapeDtypeStruct((B, 1), jnp.float32),
        compiler_params=pltpu.CompilerParams(
            vmem_limit_bytes=100 * 1024 * 1024,
        ),
        name="value_net_head",
    )(joint, *flat_b)
    return out


# speedup vs baseline: 1.2326x; 1.2326x over previous
import jax
import jax.numpy as jnp
from jax.experimental import pallas as pl
from jax.experimental.pallas import tpu as pltpu

_CELL = 3.0
_BB = 64


def _relu(x):
    return jnp.maximum(x, 0.0)


def _body_a(st_ref, xl_ref, yl_ref, xs_ref, ys_ref, self6_ref,
            w1a, b1a, gw, gx, gy, gxy, w1b, b1b, w2a, b2a, w2b, b2b,
            wa0t, wa0g, ba0, wa1, ba1, wa2v, ba2, joint_ref):
    st = st_ref[...]
    BB, N, _ = st.shape

    xk = xl_ref[...][:, None, :]
    yk = yl_ref[...][:, None, :]
    xj = xs_ref[...]
    yj = ys_ref[...]
    dx = xk - xj
    dy = yk - yj
    dist = jnp.sqrt(dx * dx + dy * dy)
    w = jnp.where(dist < _CELL, 1.0, 0.0)
    wx = jnp.where(dx > 0.0, w, 0.0)
    wy = jnp.where(dy > 0.0, w, 0.0)
    wxy = jnp.where(dy > 0.0, wx, 0.0)
    M = BB * N
    w2 = w.reshape(M, N)
    wx2 = wx.reshape(M, N)
    wy2 = wy.reshape(M, N)
    wxy2 = wxy.reshape(M, N)
    x2 = st.reshape(M, 13)

    h = _relu(jnp.dot(x2, w1a[...]) + jnp.dot(w2, gw[...])
              + jnp.dot(wx2, gx[...]) + jnp.dot(wy2, gy[...])
              + jnp.dot(wxy2, gxy[...]) + b1a[...])
    m1 = _relu(jnp.dot(h, w1b[...]) + b1b[...])
    h = _relu(jnp.dot(m1, w2a[...]) + b2a[...])
    m2 = jnp.dot(h, w2b[...]) + b2b[...]

    g = jnp.mean(m1.reshape(BB, N, 100), axis=1)
    gb = jnp.dot(g, wa0g[...])
    gb = jnp.broadcast_to(gb[:, None, :], (BB, N, 100)).reshape(BB * N, 100)
    ha = _relu(jnp.dot(m1, wa0t[...]) + gb + ba0[...])
    hb = _relu(jnp.dot(ha, wa1[...]) + ba1[...])
    scores = jnp.sum(hb.reshape(BB, N, 100) * wa2v[...][None, :, :], axis=-1)
    scores = scores + ba2[...]

    e = jnp.where(scores != 0.0, jnp.exp(scores), 0.0)
    wts = e / jnp.sum(e, axis=-1, keepdims=True)
    weighted = jnp.sum(wts[:, :, None] * m2.reshape(BB, N, 50), axis=1)

    joint_ref[...] = jnp.concatenate([self6_ref[...], weighted], axis=-1)


def _body_b(joint_ref, w30, b30, w31, b31, w32, b32, w33, b33,
            wA0, bA0, wA1, bA1, wA2, bA2, wS0, bS0, wS1, bS1, out_ref):
    j = joint_ref[...]
    v = _relu(jnp.dot(j, w30[...]) + b30[...])
    v = _relu(jnp.dot(v, w31[...]) + b31[...])
    v = _relu(jnp.dot(v, w32[...]) + b32[...])
    value = jnp.dot(v, w33[...]) + b33[...]
    a = _relu(jnp.dot(j, wA0[...]) + bA0[...])
    a = _relu(jnp.dot(a, wA1[...]) + bA1[...])
    a = jnp.dot(a, wA2[...]) + bA2[...]
    s = _relu(jnp.dot(a, wS0[...]) + bS0[...])
    adv = jnp.dot(s, wS1[...]) + bS1[...]
    out_ref[...] = value + adv - jnp.mean(adv)


def _full(w_and_b):
    W, b = w_and_b
    return [W, b.reshape(1, -1)]


def kernel(state, params):
    B, N, D = state.shape
    p1, p2, pa = params['mlp1'], params['mlp2'], params['attention']
    p3, p3a, ps = params['mlp3'], params['mlp3a'], params['adv_stream']

    xl = state[:, :, 6]
    yl = state[:, :, 7]
    xs = xl[:, :, None]
    ys = yl[:, :, None]
    self6 = state[:, 0, :6]

    wa0, ba0 = pa[0]
    w1a_full, b1a = p1[0]
    w1a = w1a_full[:13, :]
    wb = w1a_full[13:, :]
    ones = jnp.ones((N, 1), jnp.float32)
    gw = ones * wb[3:4, :]
    gx = ones * (wb[1:2, :] - wb[3:4, :])
    gy = ones * (wb[2:3, :] - wb[3:4, :])
    gxy = ones * (wb[0:1, :] - wb[1:2, :] - wb[2:3, :] + wb[3:4, :])
    b1a_adj = (b1a - wb[3, :]).reshape(1, -1)

    flat_a = ([w1a, b1a_adj, gw, gx, gy, gxy]
              + _full(p1[1]) + _full(p2[0]) + _full(p2[1])
              + [wa0[:100, :], wa0[100:, :], ba0.reshape(1, -1)]
              + _full(pa[1])
              + [pa[2][0].reshape(1, -1), pa[2][1].reshape(1, 1)])

    grid = (B // _BB,)
    full = lambda a: pl.BlockSpec(a.shape, lambda i: (0,) * a.ndim)
    joint = pl.pallas_call(
        _body_a,
        grid=grid,
        in_specs=[pl.BlockSpec((_BB, N, D), lambda i: (i, 0, 0)),
                  pl.BlockSpec((_BB, N), lambda i: (i, 0)),
                  pl.BlockSpec((_BB, N), lambda i: (i, 0)),
                  pl.BlockSpec((_BB, N, 1), lambda i: (i, 0, 0)),
                  pl.BlockSpec((_BB, N, 1), lambda i: (i, 0, 0)),
                  pl.BlockSpec((_BB, 6), lambda i: (i, 0))]
        + [full(a) for a in flat_a],
        out_specs=pl.BlockSpec((_BB, 56), lambda i: (i, 0)),
        out_shape=jax.ShapeDtypeStruct((B, 56), jnp.float32),
        compiler_params=pltpu.CompilerParams(
            dimension_semantics=("arbitrary",),
            vmem_limit_bytes=100 * 1024 * 1024,
        ),
        name="value_net_main",
    )(state, xl, yl, xs, ys, self6, *flat_a)

    flat_b = (_full(p3[0]) + _full(p3[1]) + _full(p3[2]) + _full(p3[3])
              + _full(p3a[0]) + _full(p3a[1]) + _full(p3a[2])
              + _full(ps[0]) + _full(ps[1]))
    out = pl.pallas_call(
        _body_b,
        out_shape=jax.ShapeDtypeStruct((B, 1), jnp.float32),
        compiler_params=pltpu.CompilerParams(
            vmem_limit_bytes=100 * 1024 * 1024,
        ),
        name="value_net_head",
    )(joint, *flat_b)
    return out


# speedup vs baseline: 1.2470x; 1.0117x over previous
import jax
import jax.numpy as jnp
from jax.experimental import pallas as pl
from jax.experimental.pallas import tpu as pltpu

_CELL = 3.0
_BB = 128


def _relu(x):
    return jnp.maximum(x, 0.0)


def _body_a(st_ref, xl_ref, yl_ref, xs_ref, ys_ref, self6_ref,
            w1a, b1a, gw, gx, gy, gxy, w1b, b1b, w2a, b2a, w2b, b2b,
            wa0t, wa0g, ba0, wa1, ba1, wa2v, ba2, joint_ref):
    st = st_ref[...]
    BB, N, _ = st.shape

    xk = xl_ref[...][:, None, :]
    yk = yl_ref[...][:, None, :]
    xj = xs_ref[...]
    yj = ys_ref[...]
    dx = xk - xj
    dy = yk - yj
    dist = jnp.sqrt(dx * dx + dy * dy)
    w = jnp.where(dist < _CELL, 1.0, 0.0)
    wx = jnp.where(dx > 0.0, w, 0.0)
    wy = jnp.where(dy > 0.0, w, 0.0)
    wxy = jnp.where(dy > 0.0, wx, 0.0)
    M = BB * N
    w2 = w.reshape(M, N)
    wx2 = wx.reshape(M, N)
    wy2 = wy.reshape(M, N)
    wxy2 = wxy.reshape(M, N)
    x2 = st.reshape(M, 13)

    h = _relu(jnp.dot(x2, w1a[...]) + jnp.dot(w2, gw[...])
              + jnp.dot(wx2, gx[...]) + jnp.dot(wy2, gy[...])
              + jnp.dot(wxy2, gxy[...]) + b1a[...])
    m1 = _relu(jnp.dot(h, w1b[...]) + b1b[...])
    h = _relu(jnp.dot(m1, w2a[...]) + b2a[...])
    m2 = jnp.dot(h, w2b[...]) + b2b[...]

    g = jnp.mean(m1.reshape(BB, N, 100), axis=1)
    gb = jnp.dot(g, wa0g[...])
    gb = jnp.broadcast_to(gb[:, None, :], (BB, N, 100)).reshape(BB * N, 100)
    ha = _relu(jnp.dot(m1, wa0t[...]) + gb + ba0[...])
    hb = _relu(jnp.dot(ha, wa1[...]) + ba1[...])
    scores = jnp.sum(hb.reshape(BB, N, 100) * wa2v[...][None, :, :], axis=-1)
    scores = scores + ba2[...]

    e = jnp.where(scores != 0.0, jnp.exp(scores), 0.0)
    wts = e / jnp.sum(e, axis=-1, keepdims=True)
    weighted = jnp.sum(wts[:, :, None] * m2.reshape(BB, N, 50), axis=1)

    joint_ref[...] = jnp.concatenate([self6_ref[...], weighted], axis=-1)


def _body_b(joint_ref, w30, b30, w31, b31, w32, b32, w33, b33,
            wA0, bA0, wA1, bA1, wA2, bA2, wS0, bS0, wS1, bS1, out_ref):
    j = joint_ref[...]
    v = _relu(jnp.dot(j, w30[...]) + b30[...])
    v = _relu(jnp.dot(v, w31[...]) + b31[...])
    v = _relu(jnp.dot(v, w32[...]) + b32[...])
    value = jnp.dot(v, w33[...]) + b33[...]
    a = _relu(jnp.dot(j, wA0[...]) + bA0[...])
    a = _relu(jnp.dot(a, wA1[...]) + bA1[...])
    a = jnp.dot(a, wA2[...]) + bA2[...]
    s = _relu(jnp.dot(a, wS0[...]) + bS0[...])
    adv = jnp.dot(s, wS1[...]) + bS1[...]
    out_ref[...] = value + adv - jnp.mean(adv)


def _full(w_and_b):
    W, b = w_and_b
    return [W, b.reshape(1, -1)]


def kernel(state, params):
    B, N, D = state.shape
    p1, p2, pa = params['mlp1'], params['mlp2'], params['attention']
    p3, p3a, ps = params['mlp3'], params['mlp3a'], params['adv_stream']

    xl = state[:, :, 6]
    yl = state[:, :, 7]
    xs = xl[:, :, None]
    ys = yl[:, :, None]
    self6 = state[:, 0, :6]

    wa0, ba0 = pa[0]
    w1a_full, b1a = p1[0]
    w1a = w1a_full[:13, :]
    wb = w1a_full[13:, :]
    ones = jnp.ones((N, 1), jnp.float32)
    gw = ones * wb[3:4, :]
    gx = ones * (wb[1:2, :] - wb[3:4, :])
    gy = ones * (wb[2:3, :] - wb[3:4, :])
    gxy = ones * (wb[0:1, :] - wb[1:2, :] - wb[2:3, :] + wb[3:4, :])
    b1a_adj = (b1a - wb[3, :]).reshape(1, -1)

    flat_a = ([w1a, b1a_adj, gw, gx, gy, gxy]
              + _full(p1[1]) + _full(p2[0]) + _full(p2[1])
              + [wa0[:100, :], wa0[100:, :], ba0.reshape(1, -1)]
              + _full(pa[1])
              + [pa[2][0].reshape(1, -1), pa[2][1].reshape(1, 1)])

    grid = (B // _BB,)
    full = lambda a: pl.BlockSpec(a.shape, lambda i: (0,) * a.ndim)
    joint = pl.pallas_call(
        _body_a,
        grid=grid,
        in_specs=[pl.BlockSpec((_BB, N, D), lambda i: (i, 0, 0)),
                  pl.BlockSpec((_BB, N), lambda i: (i, 0)),
                  pl.BlockSpec((_BB, N), lambda i: (i, 0)),
                  pl.BlockSpec((_BB, N, 1), lambda i: (i, 0, 0)),
                  pl.BlockSpec((_BB, N, 1), lambda i: (i, 0, 0)),
                  pl.BlockSpec((_BB, 6), lambda i: (i, 0))]
        + [full(a) for a in flat_a],
        out_specs=pl.BlockSpec((_BB, 56), lambda i: (i, 0)),
        out_shape=jax.ShapeDtypeStruct((B, 56), jnp.float32),
        compiler_params=pltpu.CompilerParams(
            dimension_semantics=("arbitrary",),
            vmem_limit_bytes=100 * 1024 * 1024,
        ),
        name="value_net_main",
    )(state, xl, yl, xs, ys, self6, *flat_a)

    flat_b = (_full(p3[0]) + _full(p3[1]) + _full(p3[2]) + _full(p3[3])
              + _full(p3a[0]) + _full(p3a[1]) + _full(p3a[2])
              + _full(ps[0]) + _full(ps[1]))
    out = pl.pallas_call(
        _body_b,
        out_shape=jax.ShapeDtypeStruct((B, 1), jnp.float32),
        compiler_params=pltpu.CompilerParams(
            vmem_limit_bytes=100 * 1024 * 1024,
        ),
        name="value_net_head",
    )(joint, *flat_b)
    return out


# speedup vs baseline: 1.5088x; 1.2099x over previous
import jax
import jax.numpy as jnp
from jax.experimental import pallas as pl
from jax.experimental.pallas import tpu as pltpu

_CELL = 3.0
_BB = 128


def _relu(x):
    return jnp.maximum(x, 0.0)


def _body_a(st_ref, xk4_ref, yk4_ref, xs_ref, ys_ref, self6_ref, cx_ref, cy_ref,
            w1a, b1a, gcat, w1b, b1b, w2a, b2a, w2b, b2b,
            wa0t, wa0g, ba0, wa1, ba1, wa2v, ba2, joint_ref):
    st = st_ref[...]
    BB, N, _ = st.shape

    xk = xk4_ref[...][:, None, :]
    yk = yk4_ref[...][:, None, :]
    xj = xs_ref[...]
    yj = ys_ref[...]
    dx = xk - xj
    dy = yk - yj
    d2 = dx * dx + dy * dy
    m = jnp.where(d2 < _CELL * _CELL, 1.0, 0.0)
    m = jnp.where(dx > cx_ref[...][:, None, :], m, 0.0)
    m = jnp.where(dy > cy_ref[...][:, None, :], m, 0.0)
    M = BB * N
    q2 = m.reshape(M, 4 * N)
    x2 = st.reshape(M, 13)

    h = _relu(jnp.dot(x2, w1a[...]) + jnp.dot(q2, gcat[...]) + b1a[...])
    m1 = _relu(jnp.dot(h, w1b[...]) + b1b[...])
    h = _relu(jnp.dot(m1, w2a[...]) + b2a[...])
    m2 = jnp.dot(h, w2b[...]) + b2b[...]

    g = jnp.mean(m1.reshape(BB, N, 100), axis=1)
    gb = jnp.dot(g, wa0g[...])
    gb = jnp.broadcast_to(gb[:, None, :], (BB, N, 100)).reshape(BB * N, 100)
    ha = _relu(jnp.dot(m1, wa0t[...]) + gb + ba0[...])
    hb = _relu(jnp.dot(ha, wa1[...]) + ba1[...])
    scores = jnp.sum(hb.reshape(BB, N, 100) * wa2v[...][None, :, :], axis=-1)
    scores = scores + ba2[...]

    e = jnp.where(scores != 0.0, jnp.exp(scores), 0.0)
    wts = e / jnp.sum(e, axis=-1, keepdims=True)
    weighted = jnp.sum(wts[:, :, None] * m2.reshape(BB, N, 50), axis=1)

    joint_ref[...] = jnp.concatenate([self6_ref[...], weighted], axis=-1)


def _body_b(joint_ref, w30, b30, w31, b31, w32, b32, w33, b33,
            wA0, bA0, wA1, bA1, wA2, bA2, wS0, bS0, wS1, bS1, out_ref):
    j = joint_ref[...]
    v = _relu(jnp.dot(j, w30[...]) + b30[...])
    v = _relu(jnp.dot(v, w31[...]) + b31[...])
    v = _relu(jnp.dot(v, w32[...]) + b32[...])
    value = jnp.dot(v, w33[...]) + b33[...]
    a = _relu(jnp.dot(j, wA0[...]) + bA0[...])
    a = _relu(jnp.dot(a, wA1[...]) + bA1[...])
    a = jnp.dot(a, wA2[...]) + bA2[...]
    s = _relu(jnp.dot(a, wS0[...]) + bS0[...])
    adv = jnp.dot(s, wS1[...]) + bS1[...]
    out_ref[...] = value + adv - jnp.mean(adv)


def _full(w_and_b):
    W, b = w_and_b
    return [W, b.reshape(1, -1)]


def kernel(state, params):
    B, N, D = state.shape
    p1, p2, pa = params['mlp1'], params['mlp2'], params['attention']
    p3, p3a, ps = params['mlp3'], params['mlp3a'], params['adv_stream']

    xl = state[:, :, 6]
    yl = state[:, :, 7]
    xk4 = jnp.concatenate([xl] * 4, axis=1)
    yk4 = jnp.concatenate([yl] * 4, axis=1)
    xs = xl[:, :, None]
    ys = yl[:, :, None]
    self6 = state[:, 0, :6]
    ninf = jnp.float32(-jnp.inf)
    zero = jnp.float32(0.0)
    cx = jnp.concatenate([jnp.full((1, N), v, jnp.float32)
                          for v in (ninf, zero, ninf, zero)], axis=1)
    cy = jnp.concatenate([jnp.full((1, N), v, jnp.float32)
                          for v in (ninf, ninf, zero, zero)], axis=1)

    wa0, ba0 = pa[0]
    w1a_full, b1a = p1[0]
    w1a = w1a_full[:13, :]
    wb = w1a_full[13:, :]
    ones = jnp.ones((N, 1), jnp.float32)
    gcat = jnp.concatenate([
        ones * wb[3:4, :],
        ones * (wb[1:2, :] - wb[3:4, :]),
        ones * (wb[2:3, :] - wb[3:4, :]),
        ones * (wb[0:1, :] - wb[1:2, :] - wb[2:3, :] + wb[3:4, :]),
    ], axis=0)
    b1a_adj = (b1a - wb[3, :]).reshape(1, -1)

    flat_a = ([w1a, b1a_adj, gcat]
              + _full(p1[1]) + _full(p2[0]) + _full(p2[1])
              + [wa0[:100, :], wa0[100:, :], ba0.reshape(1, -1)]
              + _full(pa[1])
              + [pa[2][0].reshape(1, -1), pa[2][1].reshape(1, 1)])

    grid = (B // _BB,)
    full = lambda a: pl.BlockSpec(a.shape, lambda i: (0,) * a.ndim)
    joint = pl.pallas_call(
        _body_a,
        grid=grid,
        in_specs=[pl.BlockSpec((_BB, N, D), lambda i: (i, 0, 0)),
                  pl.BlockSpec((_BB, 4 * N), lambda i: (i, 0)),
                  pl.BlockSpec((_BB, 4 * N), lambda i: (i, 0)),
                  pl.BlockSpec((_BB, N, 1), lambda i: (i, 0, 0)),
                  pl.BlockSpec((_BB, N, 1), lambda i: (i, 0, 0)),
                  pl.BlockSpec((_BB, 6), lambda i: (i, 0)),
                  full(cx), full(cy)]
        + [full(a) for a in flat_a],
        out_specs=pl.BlockSpec((_BB, 56), lambda i: (i, 0)),
        out_shape=jax.ShapeDtypeStruct((B, 56), jnp.float32),
        compiler_params=pltpu.CompilerParams(
            dimension_semantics=("arbitrary",),
            vmem_limit_bytes=100 * 1024 * 1024,
        ),
        name="value_net_main",
    )(state, xk4, yk4, xs, ys, self6, cx, cy, *flat_a)

    flat_b = (_full(p3[0]) + _full(p3[1]) + _full(p3[2]) + _full(p3[3])
              + _full(p3a[0]) + _full(p3a[1]) + _full(p3a[2])
              + _full(ps[0]) + _full(ps[1]))
    out = pl.pallas_call(
        _body_b,
        out_shape=jax.ShapeDtypeStruct((B, 1), jnp.float32),
        compiler_params=pltpu.CompilerParams(
            vmem_limit_bytes=100 * 1024 * 1024,
        ),
        name="value_net_head",
    )(joint, *flat_b)
    return out


# speedup vs baseline: 1.5286x; 1.0131x over previous
import jax
import jax.numpy as jnp
from jax.experimental import pallas as pl
from jax.experimental.pallas import tpu as pltpu

_CELL = 3.0
_BB = 128


def _relu(x):
    return jnp.maximum(x, 0.0)


def _body_a(st_ref, xk4_ref, yk4_ref, xs_ref, ys_ref, self6_ref, cx_ref, cy_ref,
            gcat, b1a, w1b, b1b, w2a, b2a, w2b, b2b,
            wa0t, wa0g, ba0, wa1, ba1, wa2v, ba2, joint_ref):
    st = st_ref[...]
    BB, N, _ = st.shape

    xk = xk4_ref[...][:, None, :]
    yk = yk4_ref[...][:, None, :]
    xj = xs_ref[...]
    yj = ys_ref[...]
    dx = xk - xj
    dy = yk - yj
    d2 = dx * dx + dy * dy
    m = jnp.where(d2 < _CELL * _CELL, 1.0, 0.0)
    m = jnp.where(dx > cx_ref[...][:, None, :], m, 0.0)
    m = jnp.where(dy > cy_ref[...][:, None, :], m, 0.0)
    M = BB * N
    q2 = jnp.concatenate([m.reshape(M, 4 * N), st.reshape(M, 13)], axis=1)

    h = _relu(jnp.dot(q2, gcat[...]) + b1a[...])
    m1 = _relu(jnp.dot(h, w1b[...]) + b1b[...])
    h2 = _relu(jnp.dot(m1, w2a[...]) + b2a[...])

    g = jnp.mean(m1.reshape(BB, N, 100), axis=1)
    gb = jnp.dot(g, wa0g[...]) + ba0[...]
    gb = jnp.broadcast_to(gb[:, None, :], (BB, N, 100)).reshape(BB * N, 100)
    ha = _relu(jnp.dot(m1, wa0t[...]) + gb)
    hb = _relu(jnp.dot(ha, wa1[...]) + ba1[...])
    scores = jnp.sum(hb.reshape(BB, N, 100) * wa2v[...][None, :, :], axis=-1)
    scores = scores + ba2[...]

    e = jnp.where(scores != 0.0, jnp.exp(scores), 0.0)
    wts = e / jnp.sum(e, axis=-1, keepdims=True)
    wh = jnp.sum(wts[:, :, None] * h2.reshape(BB, N, 100), axis=1)
    weighted = jnp.dot(wh, w2b[...]) + b2b[...]

    joint_ref[...] = jnp.concatenate([self6_ref[...], weighted], axis=-1)


def _body_b(joint_ref, w30, b30, w31, b31, w32, b32, w33, b33,
            wA0, bA0, wA1, bA1, wA2, bA2, wS0, bS0, wS1, bS1, out_ref):
    j = joint_ref[...]
    v = _relu(jnp.dot(j, w30[...]) + b30[...])
    v = _relu(jnp.dot(v, w31[...]) + b31[...])
    v = _relu(jnp.dot(v, w32[...]) + b32[...])
    value = jnp.dot(v, w33[...]) + b33[...]
    a = _relu(jnp.dot(j, wA0[...]) + bA0[...])
    a = _relu(jnp.dot(a, wA1[...]) + bA1[...])
    a = jnp.dot(a, wA2[...]) + bA2[...]
    s = _relu(jnp.dot(a, wS0[...]) + bS0[...])
    adv = jnp.dot(s, wS1[...]) + bS1[...]
    out_ref[...] = value + adv - jnp.mean(adv)


def _full(w_and_b):
    W, b = w_and_b
    return [W, b.reshape(1, -1)]


def kernel(state, params):
    B, N, D = state.shape
    p1, p2, pa = params['mlp1'], params['mlp2'], params['attention']
    p3, p3a, ps = params['mlp3'], params['mlp3a'], params['adv_stream']

    xl = state[:, :, 6]
    yl = state[:, :, 7]
    xk4 = jnp.concatenate([xl] * 4, axis=1)
    yk4 = jnp.concatenate([yl] * 4, axis=1)
    xs = xl[:, :, None]
    ys = yl[:, :, None]
    self6 = state[:, 0, :6]
    ninf = jnp.float32(-jnp.inf)
    zero = jnp.float32(0.0)
    cx = jnp.concatenate([jnp.full((1, N), v, jnp.float32)
                          for v in (ninf, zero, ninf, zero)], axis=1)
    cy = jnp.concatenate([jnp.full((1, N), v, jnp.float32)
                          for v in (ninf, ninf, zero, zero)], axis=1)

    wa0, ba0 = pa[0]
    w1a_full, b1a = p1[0]
    w1a = w1a_full[:13, :]
    wb = w1a_full[13:, :]
    ones = jnp.ones((N, 1), jnp.float32)
    gcat = jnp.concatenate([
        ones * wb[3:4, :],
        ones * (wb[1:2, :] - wb[3:4, :]),
        ones * (wb[2:3, :] - wb[3:4, :]),
        ones * (wb[0:1, :] - wb[1:2, :] - wb[2:3, :] + wb[3:4, :]),
        w1a,
    ], axis=0)
    b1a_adj = (b1a - wb[3, :]).reshape(1, -1)

    flat_a = ([gcat, b1a_adj]
              + _full(p1[1]) + _full(p2[0]) + _full(p2[1])
              + [wa0[:100, :], wa0[100:, :], ba0.reshape(1, -1)]
              + _full(pa[1])
              + [pa[2][0].reshape(1, -1), pa[2][1].reshape(1, 1)])

    grid = (B // _BB,)
    full = lambda a: pl.BlockSpec(a.shape, lambda i: (0,) * a.ndim)
    joint = pl.pallas_call(
        _body_a,
        grid=grid,
        in_specs=[pl.BlockSpec((_BB, N, D), lambda i: (i, 0, 0)),
                  pl.BlockSpec((_BB, 4 * N), lambda i: (i, 0)),
                  pl.BlockSpec((_BB, 4 * N), lambda i: (i, 0)),
                  pl.BlockSpec((_BB, N, 1), lambda i: (i, 0, 0)),
                  pl.BlockSpec((_BB, N, 1), lambda i: (i, 0, 0)),
                  pl.BlockSpec((_BB, 6), lambda i: (i, 0)),
                  full(cx), full(cy)]
        + [full(a) for a in flat_a],
        out_specs=pl.BlockSpec((_BB, 56), lambda i: (i, 0)),
        out_shape=jax.ShapeDtypeStruct((B, 56), jnp.float32),
        compiler_params=pltpu.CompilerParams(
            dimension_semantics=("arbitrary",),
            vmem_limit_bytes=100 * 1024 * 1024,
        ),
        name="value_net_main",
    )(state, xk4, yk4, xs, ys, self6, cx, cy, *flat_a)

    flat_b = (_full(p3[0]) + _full(p3[1]) + _full(p3[2]) + _full(p3[3])
              + _full(p3a[0]) + _full(p3a[1]) + _full(p3a[2])
              + _full(ps[0]) + _full(ps[1]))
    out = pl.pallas_call(
        _body_b,
        out_shape=jax.ShapeDtypeStruct((B, 1), jnp.float32),
        compiler_params=pltpu.CompilerParams(
            vmem_limit_bytes=100 * 1024 * 1024,
        ),
        name="value_net_head",
    )(joint, *flat_b)
    return out


# speedup vs baseline: 1.5550x; 1.0173x over previous
import jax
import jax.numpy as jnp
from jax.experimental import pallas as pl
from jax.experimental.pallas import tpu as pltpu

_CELL = 3.0
_BB = 128


def _relu(x):
    return jnp.maximum(x, 0.0)


def _body_a(st_ref, xk4_ref, yk4_ref, xs_ref, ys_ref, self6_ref, cx_ref, cy_ref,
            gcat, b1a, w1b, b1b, w2a, b2a, w2b, b2b,
            wa0t, wa0g, ba0, wa1, ba1, wa2v, ba2, joint_ref):
    st = st_ref[...]
    BB, N, _ = st.shape

    xk = xk4_ref[...][:, None, :]
    yk = yk4_ref[...][:, None, :]
    xj = xs_ref[...]
    yj = ys_ref[...]
    dx = xk - xj
    dy = yk - yj
    d2 = dx * dx + dy * dy
    m = jnp.where(d2 < _CELL * _CELL, 1.0, 0.0)
    m = jnp.where(dx > cx_ref[...][:, None, :], m, 0.0)
    m = jnp.where(dy > cy_ref[...][:, None, :], m, 0.0)
    M = BB * N
    q2 = jnp.concatenate([m.reshape(M, 4 * N), st.reshape(M, 13)], axis=1)

    h = _relu(jnp.dot(q2, gcat[...]) + b1a[...])
    m1 = _relu(jnp.dot(h, w1b[...]) + b1b[...])
    h2 = _relu(jnp.dot(m1, w2a[...]) + b2a[...])

    g = jnp.mean(m1.reshape(BB, N, 100), axis=1)
    gb = jnp.dot(g, wa0g[...]) + ba0[...]
    gb = jnp.broadcast_to(gb[:, None, :], (BB, N, 100)).reshape(BB * N, 100)
    ha = _relu(jnp.dot(m1, wa0t[...]) + gb)
    hb = _relu(jnp.dot(ha, wa1[...]) + ba1[...])
    scores = jnp.sum(hb.reshape(BB, N, 100) * wa2v[...][None, :, :], axis=-1)
    scores = scores + ba2[...]

    e = jnp.where(scores != 0.0, jnp.exp(scores), 0.0)
    wts = e / jnp.sum(e, axis=-1, keepdims=True)
    m2 = jnp.dot(h2, w2b[...]) + b2b[...]
    weighted = jnp.sum(wts[:, :, None] * m2.reshape(BB, N, 50), axis=1)

    joint_ref[...] = jnp.concatenate([self6_ref[...], weighted], axis=-1)


def _body_b(joint_ref, w30, b30, w31, b31, w32, b32, w33, b33,
            wA0, bA0, wA1, bA1, wA2, bA2, wS0, bS0, wS1, bS1, out_ref):
    j = joint_ref[...]
    v = _relu(jnp.dot(j, w30[...]) + b30[...])
    v = _relu(jnp.dot(v, w31[...]) + b31[...])
    v = _relu(jnp.dot(v, w32[...]) + b32[...])
    value = jnp.dot(v, w33[...]) + b33[...]
    a = _relu(jnp.dot(j, wA0[...]) + bA0[...])
    a = _relu(jnp.dot(a, wA1[...]) + bA1[...])
    a = jnp.dot(a, wA2[...]) + bA2[...]
    s = _relu(jnp.dot(a, wS0[...]) + bS0[...])
    adv = jnp.dot(s, wS1[...]) + bS1[...]
    out_ref[...] = value + adv - jnp.mean(adv)


def _full(w_and_b):
    W, b = w_and_b
    return [W, b.reshape(1, -1)]


def kernel(state, params):
    B, N, D = state.shape
    p1, p2, pa = params['mlp1'], params['mlp2'], params['attention']
    p3, p3a, ps = params['mlp3'], params['mlp3a'], params['adv_stream']

    xl = state[:, :, 6]
    yl = state[:, :, 7]
    xk4 = jnp.concatenate([xl] * 4, axis=1)
    yk4 = jnp.concatenate([yl] * 4, axis=1)
    xs = xl[:, :, None]
    ys = yl[:, :, None]
    self6 = state[:, 0, :6]
    ninf = jnp.float32(-jnp.inf)
    zero = jnp.float32(0.0)
    cx = jnp.concatenate([jnp.full((1, N), v, jnp.float32)
                          for v in (ninf, zero, ninf, zero)], axis=1)
    cy = jnp.concatenate([jnp.full((1, N), v, jnp.float32)
                          for v in (ninf, ninf, zero, zero)], axis=1)

    wa0, ba0 = pa[0]
    w1a_full, b1a = p1[0]
    w1a = w1a_full[:13, :]
    wb = w1a_full[13:, :]
    ones = jnp.ones((N, 1), jnp.float32)
    gcat = jnp.concatenate([
        ones * wb[3:4, :],
        ones * (wb[1:2, :] - wb[3:4, :]),
        ones * (wb[2:3, :] - wb[3:4, :]),
        ones * (wb[0:1, :] - wb[1:2, :] - wb[2:3, :] + wb[3:4, :]),
        w1a,
    ], axis=0)
    b1a_adj = (b1a - wb[3, :]).reshape(1, -1)

    flat_a = ([gcat, b1a_adj]
              + _full(p1[1]) + _full(p2[0]) + _full(p2[1])
              + [wa0[:100, :], wa0[100:, :], ba0.reshape(1, -1)]
              + _full(pa[1])
              + [pa[2][0].reshape(1, -1), pa[2][1].reshape(1, 1)])

    grid = (B // _BB,)
    full = lambda a: pl.BlockSpec(a.shape, lambda i: (0,) * a.ndim)
    joint = pl.pallas_call(
        _body_a,
        grid=grid,
        in_specs=[pl.BlockSpec((_BB, N, D), lambda i: (i, 0, 0)),
                  pl.BlockSpec((_BB, 4 * N), lambda i: (i, 0)),
                  pl.BlockSpec((_BB, 4 * N), lambda i: (i, 0)),
                  pl.BlockSpec((_BB, N, 1), lambda i: (i, 0, 0)),
                  pl.BlockSpec((_BB, N, 1), lambda i: (i, 0, 0)),
                  pl.BlockSpec((_BB, 6), lambda i: (i, 0)),
                  full(cx), full(cy)]
        + [full(a) for a in flat_a],
        out_specs=pl.BlockSpec((_BB, 56), lambda i: (i, 0)),
        out_shape=jax.ShapeDtypeStruct((B, 56), jnp.float32),
        compiler_params=pltpu.CompilerParams(
            dimension_semantics=("arbitrary",),
            vmem_limit_bytes=100 * 1024 * 1024,
        ),
        name="value_net_main",
    )(state, xk4, yk4, xs, ys, self6, cx, cy, *flat_a)

    flat_b = (_full(p3[0]) + _full(p3[1]) + _full(p3[2]) + _full(p3[3])
              + _full(p3a[0]) + _full(p3a[1]) + _full(p3a[2])
              + _full(ps[0]) + _full(ps[1]))
    out = pl.pallas_call(
        _body_b,
        out_shape=jax.ShapeDtypeStruct((B, 1), jnp.float32),
        compiler_params=pltpu.CompilerParams(
            vmem_limit_bytes=100 * 1024 * 1024,
        ),
        name="value_net_head",
    )(joint, *flat_b)
    return out
